# baseline (speedup 1.0000x reference)
import jax
import jax.numpy as jnp
from jax import lax
from jax.experimental import pallas as pl
from jax.experimental.pallas import tpu as pltpu
from jax.experimental.pallas import tpu_sc as plsc

D = 128
S = 8192
C = 26
B = 16384
L = 16
CHUNK = 128
NV = B // L


def _body(emb, lab, ptrs, banks, out, ptrspad,
          lab_v, src_v, rows_v, idx_src, idx_w, ptr_v, scr16,
          copy_sem, gat_sem, sct_sem):
    nc = 2
    wid = lax.axis_index("s") * nc + lax.axis_index("c")

    @pl.when(wid < C)
    def _work():
        c = wid
        NCP = 16
        CPR = S // NCP
        cps = []
        for q in range(NCP):
            cp = pltpu.make_async_copy(banks.at[pl.ds(c * S + q * CPR, CPR)],
                                       out.at[pl.ds(c * S + q * CPR, CPR)],
                                       copy_sem)
            cp.start()
            cps.append(cp)

        pltpu.sync_copy(lab, lab_v)
        pltpu.sync_copy(ptrs, ptr_v)
        cvec = jnp.full((L,), c, jnp.int32)
        pvec = plsc.load_gather(ptr_v, [cvec])
        lane = lax.iota(jnp.int32, L)

        def scan_body(v, cur):
            l16 = lab_v[pl.ds(v * L, L)]
            m = l16 == cvec
            ids = lane + v * L
            plsc.store_compressed(src_v.at[pl.ds(cur, L)], ids, mask=m)
            return cur + jnp.sum(m.astype(jnp.int32))

        cnt = lax.fori_loop(0, NV, scan_body, 0, unroll=4)

        scr16[...] = (pvec + cnt) & (S - 1)
        pltpu.sync_copy(scr16, ptrspad.at[c])

        for cp in cps:
            cp.wait()

        nch = (cnt + CHUNK - 1) // CHUNK

        def chunk_body(t, carry):
            k0 = t * CHUNK
            for t2 in range(CHUNK // L):
                j = k0 + t2 * L + lane
                je = jnp.minimum(j, cnt - 1)
                sidx = plsc.load_gather(src_v, [je])
                idx_src[pl.ds(t2 * L, L)] = sidx
                idx_w[0, pl.ds(t2 * L, L)] = c * S + ((pvec + je) & (S - 1))
            g = pltpu.make_async_copy(emb.at[idx_src], rows_v, gat_sem)
            g.start()
            g.wait()
            sc = pltpu.make_async_copy(rows_v, out.at[idx_w.at[0]], sct_sem)
            sc.start()
            sc.wait()
            return carry

        lax.fori_loop(0, nch, chunk_body, 0)


_sc_call = pl.kernel(
    _body,
    out_type=[
        jax.ShapeDtypeStruct((C * S, D), jnp.float32),
        jax.ShapeDtypeStruct((C, L), jnp.int32),
    ],
    mesh=plsc.VectorSubcoreMesh(core_axis_name="c", subcore_axis_name="s"),
    compiler_params=pltpu.CompilerParams(needs_layout_passes=False),
    scratch_types=[
        pltpu.VMEM((B,), jnp.int32),
        pltpu.VMEM((B,), jnp.int32),
        pltpu.VMEM((CHUNK, D), jnp.float32),
        pltpu.VMEM((CHUNK,), jnp.int32),
        pltpu.VMEM((1, CHUNK), jnp.int32),
        pltpu.VMEM((32,), jnp.int32),
        pltpu.VMEM((L,), jnp.int32),
        pltpu.SemaphoreType.DMA,
        pltpu.SemaphoreType.DMA,
        pltpu.SemaphoreType.DMA,
    ],
)


def kernel(embeddings, labels, banks, ptrs):
    banks_flat = banks.reshape(C * S, D)
    ptrs_pad = jnp.pad(ptrs, (0, 32 - C))
    out_flat, ptrspad = _sc_call(embeddings, labels, ptrs_pad, banks_flat)
    return out_flat.reshape(C, S, D), ptrspad[:, 0]


# speedup vs baseline: 22.7679x; 22.7679x over previous
import jax
import jax.numpy as jnp
from jax import lax
from jax.experimental import pallas as pl
from jax.experimental.pallas import tpu as pltpu
from jax.experimental.pallas import tpu_sc as plsc

D = 128
S = 8192
C = 26
B = 16384
L = 16
CHUNK = 128
NV = B // L
CC = 128
NCHUNK = S // CC
NBUF = 4

_DIAG_COPY_ONLY = False


def _body(emb, lab, ptrs, banks, out, ptrspad,
          lab_v, src_v, rows_v, idx_src, idx_w, ptr_v, scr16,
          cbufs, cin_sems, cout_sems, gat_sem, sct_sem):
    nc = 2
    wid = lax.axis_index("s") * nc + lax.axis_index("c")

    @pl.when(wid < C)
    def _work():
        c = wid

        def _cin(t, b):
            return pltpu.make_async_copy(
                banks.at[pl.ds(c * S + t * CC, CC)], cbufs[b], cin_sems[b])

        def _cout(t, b):
            return pltpu.make_async_copy(
                cbufs[b], out.at[pl.ds(c * S + t * CC, CC)], cout_sems[b])

        def run_copy():
            for b in range(NBUF):
                _cin(b, b).start()

            def cbody(g, carry):
                for b in range(NBUF):
                    t = g * NBUF + b
                    _cin(t, b).wait()
                    _cout(t, b).start()
                for b in range(NBUF):
                    t = g * NBUF + b
                    _cout(t, b).wait()

                    @pl.when(g < NCHUNK // NBUF - 1)
                    def _next():
                        _cin(t + NBUF, b).start()

                return carry

            lax.fori_loop(0, NCHUNK // NBUF, cbody, 0)

        if _DIAG_COPY_ONLY:
            run_copy()
            scr16[...] = jnp.zeros((L,), jnp.int32)
            pltpu.sync_copy(scr16, ptrspad.at[c])
            return

        pltpu.sync_copy(lab, lab_v)
        pltpu.sync_copy(ptrs, ptr_v)
        cvec = jnp.full((L,), c, jnp.int32)
        pvec = plsc.load_gather(ptr_v, [cvec])
        lane = lax.iota(jnp.int32, L)

        def scan_body(v, cur):
            l16 = lab_v[pl.ds(v * L, L)]
            m = l16 == cvec
            ids = lane + v * L
            plsc.store_compressed(src_v.at[pl.ds(cur, L)], ids, mask=m)
            return cur + jnp.sum(m.astype(jnp.int32))

        cnt = lax.fori_loop(0, NV, scan_body, 0, unroll=4)

        scr16[...] = (pvec + cnt) & (S - 1)
        pltpu.sync_copy(scr16, ptrspad.at[c])

        run_copy()

        nch = (cnt + CHUNK - 1) // CHUNK

        def chunk_body(t, carry):
            k0 = t * CHUNK
            for t2 in range(CHUNK // L):
                j = k0 + t2 * L + lane
                je = jnp.minimum(j, cnt - 1)
                sidx = plsc.load_gather(src_v, [je])
                idx_src[pl.ds(t2 * L, L)] = sidx
                idx_w[0, pl.ds(t2 * L, L)] = c * S + ((pvec + je) & (S - 1))
            g = pltpu.make_async_copy(emb.at[idx_src], rows_v, gat_sem)
            g.start()
            g.wait()
            sc = pltpu.make_async_copy(rows_v, out.at[idx_w.at[0]], sct_sem)
            sc.start()
            sc.wait()
            return carry

        lax.fori_loop(0, nch, chunk_body, 0)


_sc_call = pl.kernel(
    _body,
    out_type=[
        jax.ShapeDtypeStruct((C * S, D), jnp.float32),
        jax.ShapeDtypeStruct((C, L), jnp.int32),
    ],
    mesh=plsc.VectorSubcoreMesh(core_axis_name="c", subcore_axis_name="s"),
    compiler_params=pltpu.CompilerParams(needs_layout_passes=False),
    scratch_types=[
        pltpu.VMEM((B,), jnp.int32),
        pltpu.VMEM((B,), jnp.int32),
        pltpu.VMEM((CHUNK, D), jnp.float32),
        pltpu.VMEM((CHUNK,), jnp.int32),
        pltpu.VMEM((1, CHUNK), jnp.int32),
        pltpu.VMEM((32,), jnp.int32),
        pltpu.VMEM((L,), jnp.int32),
        [pltpu.VMEM((CC, D), jnp.float32) for _ in range(NBUF)],
        [pltpu.SemaphoreType.DMA for _ in range(NBUF)],
        [pltpu.SemaphoreType.DMA for _ in range(NBUF)],
        pltpu.SemaphoreType.DMA,
        pltpu.SemaphoreType.DMA,
    ],
)


def kernel(embeddings, labels, banks, ptrs):
    banks_flat = banks.reshape(C * S, D)
    ptrs_pad = jnp.pad(ptrs, (0, 32 - C))
    out_flat, ptrspad = _sc_call(embeddings, labels, ptrs_pad, banks_flat)
    return out_flat.reshape(C, S, D), ptrspad[:, 0]


# speedup vs baseline: 26.6278x; 1.1695x over previous
import jax
import jax.numpy as jnp
from jax import lax
from jax.experimental import pallas as pl
from jax.experimental.pallas import tpu as pltpu
from jax.experimental.pallas import tpu_sc as plsc

D = 128
S = 8192
C = 26
B = 16384
L = 16
CHUNK = 128
NV = B // L
CC = 128
NBUF = 4
KLAG = 2
NG = 13
CPW = NG * NBUF
SPC = S // L // NBUF * NBUF
ROWS_PER_SUB = S // 16


def _body(emb, lab, ptrs, banks, out, ptrspad,
          lab_v, src_v, idx_src, idx_w, ptr_v, scr16,
          cbufs, cin_sems, cout_sems):
    ncores = 2
    core = lax.axis_index("c")
    sub = lax.axis_index("s")
    wid = sub * ncores + core

    pltpu.sync_copy(lab, lab_v)
    pltpu.sync_copy(ptrs, ptr_v)
    cvec = jnp.full((L,), wid, jnp.int32)
    pvec = plsc.load_gather(ptr_v, [jnp.minimum(cvec, C - 1)])
    lane = lax.iota(jnp.int32, L)

    def _cbase(g, off):
        q, r = divmod(off, NBUF)
        return (2 * (g + q) + core) * S + sub * ROWS_PER_SUB + r * CC

    def _cin(g, off, b):
        return pltpu.make_async_copy(
            banks.at[pl.ds(_cbase(g, off), CC)], cbufs[b], cin_sems[b])

    def _cout(g, off, b):
        return pltpu.make_async_copy(
            cbufs[b], out.at[pl.ds(_cbase(g, off), CC)], cout_sems[b])

    def scan_body(v, cur):
        l16 = lab_v[pl.ds(v * L, L)]
        m = l16 == cvec
        ids = lane + v * L
        plsc.store_compressed(src_v.at[pl.ds(cur, L)], ids, mask=m)
        return cur + jnp.sum(m.astype(jnp.int32))

    _cin(0, 0, 0).start()
    _cin(0, 1, 1).start()

    def cbody(g, cur):
        cur = lax.fori_loop(g * NV // NG, (g + 1) * NV // NG,
                            scan_body, cur)
        for b in range(NBUF):
            t = g * NBUF + b
            _cin(g, b, b).wait()
            _cout(g, b, b).start()
            if b >= KLAG:
                _cout(g, b - KLAG, b - KLAG).wait()
            else:
                @pl.when(g > 0)
                def _wout():
                    _cout(g - 1, b - KLAG + NBUF, b - KLAG + NBUF).wait()

            @pl.when(t + KLAG < CPW)
            def _nin():
                _cin(g, b + KLAG, (b + KLAG) % NBUF).start()

        return cur

    cnt = lax.fori_loop(0, NG, cbody, 0)
    _cout(NG - 1, NBUF - KLAG, NBUF - KLAG).wait()
    _cout(NG - 1, NBUF - 1, NBUF - 1).wait()

    @pl.when(wid < C)
    def _wptr():
        scr16[...] = (pvec + cnt) & (S - 1)
        pltpu.sync_copy(scr16, ptrspad.at[wid])

    plsc.subcore_barrier()

    @pl.when(wid < C)
    def _scatter():
        c = wid
        nch = (cnt + CHUNK - 1) // CHUNK

        def build_idx(t, b):
            k0 = t * CHUNK
            for t2 in range(CHUNK // L):
                j = k0 + t2 * L + lane
                je = jnp.minimum(j, cnt - 1)
                sidx = plsc.load_gather(src_v, [je])
                idx_src[b, pl.ds(t2 * L, L)] = sidx
                idx_w[b, pl.ds(t2 * L, L)] = c * S + ((pvec + je) & (S - 1))

        def _gat(b):
            return pltpu.make_async_copy(
                emb.at[idx_src.at[b]], cbufs[b], cin_sems[b])

        def _sct(b):
            return pltpu.make_async_copy(
                cbufs[b], out.at[idx_w.at[b]], cout_sems[b])

        @pl.when(nch > 0)
        def _prol():
            build_idx(0, 0)
            _gat(0).start()

        def pair_body(g, carry):
            for b in range(2):
                t = g * 2 + b

                @pl.when(t < nch)
                def _step():
                    _gat(b).wait()

                    @pl.when(t + 1 < nch)
                    def _nxt():
                        build_idx(t + 1, 1 - b)
                        _gat(1 - b).start()

                    _sct(b).start()
                    _sct(b).wait()

            return carry

        lax.fori_loop(0, (nch + 1) // 2, pair_body, 0)


_sc_call = pl.kernel(
    _body,
    out_type=[
        jax.ShapeDtypeStruct((C * S, D), jnp.float32),
        jax.ShapeDtypeStruct((C, L), jnp.int32),
    ],
    mesh=plsc.VectorSubcoreMesh(core_axis_name="c", subcore_axis_name="s"),
    compiler_params=pltpu.CompilerParams(needs_layout_passes=False),
    scratch_types=[
        pltpu.VMEM((B,), jnp.int32),
        pltpu.VMEM((B,), jnp.int32),
        pltpu.VMEM((2, CHUNK), jnp.int32),
        pltpu.VMEM((2, CHUNK), jnp.int32),
        pltpu.VMEM((32,), jnp.int32),
        pltpu.VMEM((L,), jnp.int32),
        [pltpu.VMEM((CC, D), jnp.float32) for _ in range(NBUF)],
        [pltpu.SemaphoreType.DMA for _ in range(NBUF)],
        [pltpu.SemaphoreType.DMA for _ in range(NBUF)],
    ],
)


def kernel(embeddings, labels, banks, ptrs):
    banks_flat = banks.reshape(C * S, D)
    ptrs_pad = jnp.pad(ptrs, (0, 32 - C))
    out_flat, ptrspad = _sc_call(embeddings, labels, ptrs_pad, banks_flat)
    return out_flat.reshape(C, S, D), ptrspad[:, 0]


# speedup vs baseline: 26.6879x; 1.0023x over previous
import jax
import jax.numpy as jnp
from jax import lax
from jax.experimental import pallas as pl
from jax.experimental.pallas import tpu as pltpu
from jax.experimental.pallas import tpu_sc as plsc

D = 128
S = 8192
C = 26
B = 16384
L = 16
CHUNK = 128
NV = B // L
CC = 128
NBUF = 4
KLAG = 2
NG = 13
CPW = NG * NBUF
SPC = S // L // NBUF * NBUF
ROWS_PER_SUB = S // 16


def _body(emb, lab, ptrs, banks, out, ptrspad,
          lab_v, src_v, idx_src, idx_w, ptr_v, scr16, cnt_v,
          cbufs, cin_sems, cout_sems):
    ncores = 2
    core = lax.axis_index("c")
    sub = lax.axis_index("s")
    wid = sub * ncores + core

    pltpu.sync_copy(lab, lab_v)
    pltpu.sync_copy(ptrs, ptr_v)
    cvec = jnp.full((L,), wid, jnp.int32)
    pvec = plsc.load_gather(ptr_v, [jnp.minimum(cvec, C - 1)])
    lane = lax.iota(jnp.int32, L)

    def _cbase(g, off):
        q, r = divmod(off, NBUF)
        return (2 * (g + q) + core) * S + sub * ROWS_PER_SUB + r * CC

    def _cin(buf, g, off, b):
        return pltpu.make_async_copy(
            banks.at[pl.ds(_cbase(g, off), CC)], buf(b), cin_sems[b])

    def _cout(buf, g, off, b):
        return pltpu.make_async_copy(
            buf(b), out.at[pl.ds(_cbase(g, off), CC)], cout_sems[b])

    def scan_body(v, cur):
        l16 = lab_v[pl.ds(v * L, L)]
        m = l16 == cvec
        ids = lane + v * L
        plsc.store_compressed(src_v.at[pl.ds(cur, L)], ids, mask=m)
        return cur + jnp.sum(m.astype(jnp.int32))

    def run_ring(buf):
        _cin(buf, 0, 0, 0).start()
        _cin(buf, 0, 1, 1).start()

        def cbody(g, cur):
            cur = lax.fori_loop(g * NV // NG, (g + 1) * NV // NG,
                                scan_body, cur)
            for b in range(NBUF):
                t = g * NBUF + b
                _cin(buf, g, b, b).wait()
                _cout(buf, g, b, b).start()
                if b >= KLAG:
                    _cout(buf, g, b - KLAG, b - KLAG).wait()
                else:
                    @pl.when(g > 0)
                    def _wout():
                        _cout(buf, g - 1, b - KLAG + NBUF,
                              b - KLAG + NBUF).wait()

                @pl.when(t + KLAG < CPW)
                def _nin():
                    _cin(buf, g, b + KLAG, (b + KLAG) % NBUF).start()

            return cur

        cnt = lax.fori_loop(0, NG, cbody, 0)
        _cout(buf, NG - 1, NBUF - KLAG, NBUF - KLAG).wait()
        _cout(buf, NG - 1, NBUF - 1, NBUF - 1).wait()
        cnt_v[...] = jnp.full((L,), cnt, jnp.int32)

    run_ring(lambda b: cbufs[b])
    cnt = cnt_v[...][0]

    @pl.when(wid < C)
    def _wptr():
        scr16[...] = (pvec + cnt) & (S - 1)
        pltpu.sync_copy(scr16, ptrspad.at[wid])

    plsc.subcore_barrier()

    @pl.when(wid < C)
    def _scatter():
        c = wid
        nch = (cnt + CHUNK - 1) // CHUNK

        def build_idx(t, b):
            k0 = t * CHUNK
            for t2 in range(CHUNK // L):
                j = k0 + t2 * L + lane
                je = jnp.minimum(j, cnt - 1)
                sidx = plsc.load_gather(src_v, [je])
                idx_src[b, pl.ds(t2 * L, L)] = sidx
                idx_w[b, pl.ds(t2 * L, L)] = c * S + ((pvec + je) & (S - 1))

        def _gat(b):
            return pltpu.make_async_copy(
                emb.at[idx_src.at[b]], cbufs[b], cin_sems[b])

        def _sct(b):
            return pltpu.make_async_copy(
                cbufs[b], out.at[idx_w.at[b]], cout_sems[b])

        @pl.when(nch > 0)
        def _prol():
            build_idx(0, 0)
            _gat(0).start()

        def pair_body(g, carry):
            for b in range(2):
                t = g * 2 + b

                @pl.when(t < nch)
                def _step():
                    _gat(b).wait()

                    @pl.when(t + 1 < nch)
                    def _nxt():
                        build_idx(t + 1, 1 - b)
                        _gat(1 - b).start()

                    _sct(b).start()
                    _sct(b).wait()

            return carry

        lax.fori_loop(0, (nch + 1) // 2, pair_body, 0)


_sc_call = pl.kernel(
    _body,
    out_type=[
        jax.ShapeDtypeStruct((C * S, D), jnp.float32),
        jax.ShapeDtypeStruct((C, L), jnp.int32),
    ],
    mesh=plsc.VectorSubcoreMesh(core_axis_name="c", subcore_axis_name="s"),
    compiler_params=pltpu.CompilerParams(needs_layout_passes=False),
    scratch_types=[
        pltpu.VMEM((B,), jnp.int32),
        pltpu.VMEM((B,), jnp.int32),
        pltpu.VMEM((2, CHUNK), jnp.int32),
        pltpu.VMEM((2, CHUNK), jnp.int32),
        pltpu.VMEM((32,), jnp.int32),
        pltpu.VMEM((L,), jnp.int32),
        pltpu.VMEM((L,), jnp.int32),
        [pltpu.VMEM((CC, D), jnp.float32) for _ in range(NBUF)],
        [pltpu.SemaphoreType.DMA for _ in range(NBUF)],
        [pltpu.SemaphoreType.DMA for _ in range(NBUF)],
    ],
)


def kernel(embeddings, labels, banks, ptrs):
    banks_flat = banks.reshape(C * S, D)
    ptrs_pad = jnp.pad(ptrs, (0, 32 - C))
    out_flat, ptrspad = _sc_call(embeddings, labels, ptrs_pad, banks_flat)
    return out_flat.reshape(C, S, D), ptrspad[:, 0]


# speedup vs baseline: 27.6623x; 1.0365x over previous
import jax
import jax.numpy as jnp
from jax import lax
from jax.experimental import pallas as pl
from jax.experimental.pallas import tpu as pltpu
from jax.experimental.pallas import tpu_sc as plsc

D = 128
S = 8192
C = 26
B = 16384
L = 16
CHUNK = 64
NV = B // L
CC = 64
NBUF = 8
KLAG = 4
NG = 13
CPW = NG * NBUF
SPC = S // L // NBUF * NBUF
ROWS_PER_SUB = S // 16


def _body(emb, lab, ptrs, banks, out, ptrspad,
          lab_v, src_v, idx_src, idx_w, ptr_v, scr16, cnt_v,
          cbufs, cin_sems, cout_sems):
    ncores = 2
    core = lax.axis_index("c")
    sub = lax.axis_index("s")
    wid = sub * ncores + core

    pltpu.sync_copy(lab, lab_v)
    pltpu.sync_copy(ptrs, ptr_v)
    cvec = jnp.full((L,), wid, jnp.int32)
    pvec = plsc.load_gather(ptr_v, [jnp.minimum(cvec, C - 1)])
    lane = lax.iota(jnp.int32, L)

    def _cbase(g, off):
        q, r = divmod(off, NBUF)
        return (2 * (g + q) + core) * S + sub * ROWS_PER_SUB + r * CC

    def _cin(buf, g, off, b):
        return pltpu.make_async_copy(
            banks.at[pl.ds(_cbase(g, off), CC)], buf(b), cin_sems[b])

    def _cout(buf, g, off, b):
        return pltpu.make_async_copy(
            buf(b), out.at[pl.ds(_cbase(g, off), CC)], cout_sems[b])

    def scan_body(v, cur):
        l16 = lab_v[pl.ds(v * L, L)]
        m = l16 == cvec
        ids = lane + v * L
        plsc.store_compressed(src_v.at[pl.ds(cur, L)], ids, mask=m)
        return cur + jnp.sum(m.astype(jnp.int32))

    def run_ring(buf):
        for i in range(KLAG):
            _cin(buf, 0, i, i).start()

        def cbody(g, cur):
            cur = lax.fori_loop(g * NV // NG, (g + 1) * NV // NG,
                                scan_body, cur)
            for b in range(NBUF):
                t = g * NBUF + b
                _cin(buf, g, b, b).wait()
                _cout(buf, g, b, b).start()
                if b >= KLAG:
                    _cout(buf, g, b - KLAG, b - KLAG).wait()
                else:
                    @pl.when(g > 0)
                    def _wout():
                        _cout(buf, g - 1, b - KLAG + NBUF,
                              b - KLAG + NBUF).wait()

                @pl.when(t + KLAG < CPW)
                def _nin():
                    _cin(buf, g, b + KLAG, (b + KLAG) % NBUF).start()

            return cur

        cnt = lax.fori_loop(0, NG, cbody, 0)
        for i in range(NBUF - KLAG, NBUF):
            _cout(buf, NG - 1, i, i).wait()
        cnt_v[...] = jnp.full((L,), cnt, jnp.int32)

    run_ring(lambda b: cbufs[b])
    cnt = cnt_v[...][0]

    @pl.when(wid < C)
    def _wptr():
        scr16[...] = (pvec + cnt) & (S - 1)
        pltpu.sync_copy(scr16, ptrspad.at[wid])

    plsc.subcore_barrier()

    @pl.when(wid < C)
    def _scatter():
        c = wid
        nch = (cnt + CHUNK - 1) // CHUNK

        def build_idx(t, b):
            k0 = t * CHUNK
            for t2 in range(CHUNK // L):
                j = k0 + t2 * L + lane
                je = jnp.minimum(j, cnt - 1)
                sidx = plsc.load_gather(src_v, [je])
                idx_src[b, pl.ds(t2 * L, L)] = sidx
                idx_w[b, pl.ds(t2 * L, L)] = c * S + ((pvec + je) & (S - 1))

        def _gat(b):
            return pltpu.make_async_copy(
                emb.at[idx_src.at[b]], cbufs[b], cin_sems[b])

        def _sct(b):
            return pltpu.make_async_copy(
                cbufs[b], out.at[idx_w.at[b]], cout_sems[b])

        @pl.when(nch > 0)
        def _prol():
            build_idx(0, 0)
            _gat(0).start()

        def pair_body(g, carry):
            for b in range(2):
                t = g * 2 + b

                @pl.when(t < nch)
                def _step():
                    _gat(b).wait()

                    @pl.when(t + 1 < nch)
                    def _nxt():
                        build_idx(t + 1, 1 - b)
                        _gat(1 - b).start()

                    _sct(b).start()
                    _sct(b).wait()

            return carry

        lax.fori_loop(0, (nch + 1) // 2, pair_body, 0)


_sc_call = pl.kernel(
    _body,
    out_type=[
        jax.ShapeDtypeStruct((C * S, D), jnp.float32),
        jax.ShapeDtypeStruct((C, L), jnp.int32),
    ],
    mesh=plsc.VectorSubcoreMesh(core_axis_name="c", subcore_axis_name="s"),
    compiler_params=pltpu.CompilerParams(needs_layout_passes=False),
    scratch_types=[
        pltpu.VMEM((B,), jnp.int32),
        pltpu.VMEM((B,), jnp.int32),
        pltpu.VMEM((2, CHUNK), jnp.int32),
        pltpu.VMEM((2, CHUNK), jnp.int32),
        pltpu.VMEM((32,), jnp.int32),
        pltpu.VMEM((L,), jnp.int32),
        pltpu.VMEM((L,), jnp.int32),
        [pltpu.VMEM((CC, D), jnp.float32) for _ in range(NBUF)],
        [pltpu.SemaphoreType.DMA for _ in range(NBUF)],
        [pltpu.SemaphoreType.DMA for _ in range(NBUF)],
    ],
)


def kernel(embeddings, labels, banks, ptrs):
    banks_flat = banks.reshape(C * S, D)
    ptrs_pad = jnp.pad(ptrs, (0, 32 - C))
    out_flat, ptrspad = _sc_call(embeddings, labels, ptrs_pad, banks_flat)
    return out_flat.reshape(C, S, D), ptrspad[:, 0]
